# Initial kernel scaffold; baseline (speedup 1.0000x reference)
import functools

import jax
import jax.numpy as jnp
from jax import lax
from jax.experimental import pallas as pl
from jax.experimental.pallas import tpu as pltpu
from jax.experimental.pallas import tpu_sc as plsc


def _proj_kernel(x_ref, wl_ref, wr_ref, b_ref, y_ref, z_ref):
    x = x_ref[...]
    y_ref[...] = jnp.dot(x, wl_ref[...], preferred_element_type=jnp.float32)
    z_ref[...] = (
        jnp.dot(x, wr_ref[...], preferred_element_type=jnp.float32) + b_ref[...]
    )


@functools.cache
def _make_segsum(n, e, c, nw, nc, epw):
    mesh = plsc.VectorSubcoreMesh(core_axis_name="c", subcore_axis_name="s")

    @functools.partial(
        pl.kernel,
        mesh=mesh,
        out_type=(
            jax.ShapeDtypeStruct((nw, c * n), jnp.float32),
            jax.ShapeDtypeStruct((nw, n), jnp.float32),
        ),
        scratch_types=[
            pltpu.VMEM((epw,), jnp.int32),
            pltpu.VMEM((epw,), jnp.int32),
            pltpu.VMEM((c * n,), jnp.float32),
            pltpu.VMEM((c * n,), jnp.float32),
            pltpu.VMEM((n,), jnp.float32),
        ],
    )
    def seg(src_hbm, dst_hbm, y_hbm, psum_hbm, pcnt_hbm,
            src_v, dst_v, y_v, acc_v, cnt_v):
        wid = lax.axis_index("s") * nc + lax.axis_index("c")
        base = wid * epw
        pltpu.sync_copy(src_hbm.at[pl.ds(base, epw)], src_v)
        pltpu.sync_copy(dst_hbm.at[pl.ds(base, epw)], dst_v)
        pltpu.sync_copy(y_hbm, y_v)

        zf = jnp.zeros((16,), jnp.float32)

        def zero_acc(i, carry):
            acc_v[pl.ds(i * 16, 16)] = zf
            return carry

        lax.fori_loop(0, (c * n) // 16, zero_acc, 0, unroll=8)

        def zero_cnt(i, carry):
            cnt_v[pl.ds(i * 16, 16)] = zf
            return carry

        lax.fori_loop(0, n // 16, zero_cnt, 0, unroll=8)

        ones = jnp.full((16,), 1.0, jnp.float32)

        def body(i, carry):
            b0 = i * 16
            s = src_v[pl.ds(b0, 16)]
            d = dst_v[pl.ds(b0, 16)]
            sc = s * c
            for j in range(c):
                yj = plsc.load_gather(y_v, [sc + j])
                plsc.addupdate_scatter(acc_v, [d + j * n], yj)
            plsc.addupdate_scatter(cnt_v, [d], ones)
            return carry

        lax.fori_loop(0, epw // 16, body, 0, unroll=4)

        pltpu.sync_copy(acc_v, psum_hbm.at[wid])
        pltpu.sync_copy(cnt_v, pcnt_hbm.at[wid])

    return seg


def _make_final(n, c):
    def _final_kernel(ps_ref, pc_ref, zt_ref, ls_ref, out_ref):
        s = jnp.sum(ps_ref[...], axis=0)
        cnt = jnp.sum(pc_ref[...], axis=0, keepdims=True)
        agg = s / jnp.maximum(cnt, 1.0)
        out_t = agg + zt_ref[...]
        m = jnp.max(out_t, axis=0, keepdims=True)
        lse = m + jnp.log(jnp.sum(jnp.exp(out_t - m), axis=0, keepdims=True))
        ls_ref[...] = out_t - lse
        out_ref[...] = out_t

    return _final_kernel


def kernel(x, edge_index, W_l, W_r, b):
    n, d = x.shape
    e = edge_index.shape[1]
    c = W_l.shape[1]

    y, z = pl.pallas_call(
        _proj_kernel,
        out_shape=(
            jax.ShapeDtypeStruct((n, c), jnp.float32),
            jax.ShapeDtypeStruct((n, c), jnp.float32),
        ),
    )(x, W_l, W_r, b.reshape(1, c))

    nc, ns = 2, 16
    nw = nc * ns
    epw = e // nw
    seg = _make_segsum(n, e, c, nw, nc, epw)
    y_flat = y.reshape(c * n)
    psum, pcnt = seg(edge_index[0], edge_index[1], y_flat)

    ls_t, out_t = pl.pallas_call(
        _make_final(n, c),
        out_shape=(
            jax.ShapeDtypeStruct((c, n), jnp.float32),
            jax.ShapeDtypeStruct((c, n), jnp.float32),
        ),
    )(psum.reshape(nw, c, n), pcnt, z.T)

    return (ls_t.T, out_t.T)


# speedup vs baseline: 29.0922x; 29.0922x over previous
import functools

import jax
import jax.numpy as jnp
from jax import lax
from jax.experimental import pallas as pl
from jax.experimental.pallas import tpu as pltpu
from jax.experimental.pallas import tpu_sc as plsc


def _proj_kernel(x_ref, wl_ref, wr_ref, b_ref, y_ref, z_ref):
    x = x_ref[...]
    y_ref[...] = jnp.dot(x, wl_ref[...], preferred_element_type=jnp.float32)
    z_ref[...] = (
        jnp.dot(x, wr_ref[...], preferred_element_type=jnp.float32) + b_ref[...]
    )


@functools.cache
def _make_segsum(n, e, c, nw, nc, epw):
    mesh = plsc.VectorSubcoreMesh(core_axis_name="c", subcore_axis_name="s")

    @functools.partial(
        pl.kernel,
        mesh=mesh,
        compiler_params=pltpu.CompilerParams(needs_layout_passes=False),
        out_type=(
            jax.ShapeDtypeStruct((nw, c * n), jnp.float32),
            jax.ShapeDtypeStruct((nw, n), jnp.float32),
        ),
        scratch_types=[
            pltpu.VMEM((epw,), jnp.int32),
            pltpu.VMEM((epw,), jnp.int32),
            pltpu.VMEM((c * n,), jnp.float32),
            pltpu.VMEM((c * n,), jnp.float32),
            pltpu.VMEM((n,), jnp.float32),
        ],
    )
    def seg(src_hbm, dst_hbm, y_hbm, psum_hbm, pcnt_hbm,
            src_v, dst_v, y_v, acc_v, cnt_v):
        wid = lax.axis_index("s") * nc + lax.axis_index("c")
        base = wid * epw
        pltpu.sync_copy(src_hbm.at[pl.ds(base, epw)], src_v)
        pltpu.sync_copy(dst_hbm.at[pl.ds(base, epw)], dst_v)
        pltpu.sync_copy(y_hbm, y_v)

        zf = jnp.zeros((16,), jnp.float32)

        def zero_acc(i, carry):
            acc_v[pl.ds(i * 16, 16)] = zf
            return carry

        lax.fori_loop(0, (c * n) // 16, zero_acc, 0, unroll=8)

        def zero_cnt(i, carry):
            cnt_v[pl.ds(i * 16, 16)] = zf
            return carry

        lax.fori_loop(0, n // 16, zero_cnt, 0, unroll=8)

        ones = jnp.full((16,), 1.0, jnp.float32)

        def body(i, carry):
            b0 = i * 16
            s = src_v[pl.ds(b0, 16)]
            d = dst_v[pl.ds(b0, 16)]
            sc = s * c
            for j in range(c):
                yj = plsc.load_gather(y_v, [sc + j])
                plsc.addupdate_scatter(acc_v, [d + j * n], yj)
            plsc.addupdate_scatter(cnt_v, [d], ones)
            return carry

        lax.fori_loop(0, epw // 16, body, 0, unroll=4)

        pltpu.sync_copy(acc_v, psum_hbm.at[wid])
        pltpu.sync_copy(cnt_v, pcnt_hbm.at[wid])

    return seg


def _make_final(n, c):
    def _final_kernel(ps_ref, pc_ref, zt_ref, ls_ref, out_ref):
        s = jnp.sum(ps_ref[...], axis=0)
        cnt = jnp.sum(pc_ref[...], axis=0, keepdims=True)
        agg = s / jnp.maximum(cnt, 1.0)
        out_t = agg + zt_ref[...]
        m = jnp.max(out_t, axis=0, keepdims=True)
        lse = m + jnp.log(jnp.sum(jnp.exp(out_t - m), axis=0, keepdims=True))
        ls_ref[...] = out_t - lse
        out_ref[...] = out_t

    return _final_kernel


def kernel(x, edge_index, W_l, W_r, b):
    n, d = x.shape
    e = edge_index.shape[1]
    c = W_l.shape[1]

    y, z = pl.pallas_call(
        _proj_kernel,
        out_shape=(
            jax.ShapeDtypeStruct((n, c), jnp.float32),
            jax.ShapeDtypeStruct((n, c), jnp.float32),
        ),
    )(x, W_l, W_r, b.reshape(1, c))

    nc, ns = 2, 16
    nw = nc * ns
    epw = e // nw
    seg = _make_segsum(n, e, c, nw, nc, epw)
    y_flat = y.reshape(c * n)
    psum, pcnt = seg(edge_index[0], edge_index[1], y_flat)

    ls_t, out_t = pl.pallas_call(
        _make_final(n, c),
        out_shape=(
            jax.ShapeDtypeStruct((c, n), jnp.float32),
            jax.ShapeDtypeStruct((c, n), jnp.float32),
        ),
    )(psum.reshape(nw, c, n), pcnt, z.T)

    return (ls_t.T, out_t.T)
